# baseline (speedup 1.0000x reference)
import functools

import jax
import jax.numpy as jnp
from jax import lax
from jax.experimental import pallas as pl
from jax.experimental.pallas import tpu as pltpu
from jax.experimental.pallas import tpu_sc as plsc

N = 100000
ATOM_ID_DIM = 32
CHARGE_DIM = 8
MOTIF_FEAT_SIZE = 48
MOTIF_DIM = 32
NUM_JOINS = 3
OUT_DIM = 232
NSLOT = 8
GDIM = NSLOT * 16

CREP = 2048
MREP = 256
SHAPE_OFS = 3 * CREP
MULT_OFS = SHAPE_OFS + 5001


CHUNK = 1000
NUM_CHUNKS = N // CHUNK
CHALF = CHUNK // 2


def _sc_gather(aidx, idx8, atab, smtab):
    info = plsc.get_sparse_core_info()
    nc, ns = info.num_cores, info.num_subcores
    nw = nc * ns
    chunks_per_w = -(-NUM_CHUNKS // nw)
    mesh = plsc.VectorSubcoreMesh(core_axis_name="c", subcore_axis_name="s")

    @functools.partial(
        pl.kernel,
        mesh=mesh,
        out_type=(
            jax.ShapeDtypeStruct((N, ATOM_ID_DIM), jnp.float32),
            jax.ShapeDtypeStruct((NSLOT * N, 16), jnp.float32),
        ),
        scratch_types=[
            pltpu.VMEM((CHUNK,), jnp.int32),
            pltpu.VMEM((NSLOT * CHALF,), jnp.int32),
            pltpu.VMEM((CHUNK, ATOM_ID_DIM), jnp.float32),
            pltpu.VMEM((NSLOT * CHALF, 16), jnp.float32),
            pltpu.SemaphoreType.DMA,
        ],
        compiler_params=pltpu.CompilerParams(use_tc_tiling_on_sc=False),
    )
    def k(aidx_hbm, idx8_hbm, atab_hbm, smtab_hbm, ga_out, gsm_out,
          aidx_v, idx_v, arows_v, grows_v, sem):
        wid = lax.axis_index("s") * nc + lax.axis_index("c")
        for c in range(chunks_per_w):
            cid = wid + nw * c

            @pl.when(cid < NUM_CHUNKS)
            def _():
                base = cid * CHUNK
                rows = pl.ds(base, CHUNK)
                pltpu.sync_copy(aidx_hbm.at[rows], aidx_v)
                pltpu.async_copy(atab_hbm.at[aidx_v], arows_v, sem).wait()
                pltpu.sync_copy(arows_v, ga_out.at[rows])
                for h in range(CHUNK // CHALF):
                    gr = pl.ds((base + h * CHALF) * NSLOT, NSLOT * CHALF)
                    pltpu.sync_copy(idx8_hbm.at[gr], idx_v)
                    pltpu.async_copy(smtab_hbm.at[idx_v], grows_v, sem).wait()
                    pltpu.sync_copy(grows_v, gsm_out.at[gr])

    return k(aidx, idx8, atab, smtab)


BR = 1000


def _tc_body(ga_ref, gsm_ref, mf_ref, pa_ref, psm_ref, pw_ref, b_ref, out_ref):
    acc = jnp.dot(ga_ref[...], pa_ref[...], preferred_element_type=jnp.float32)
    acc += jnp.dot(gsm_ref[...], psm_ref[...], preferred_element_type=jnp.float32)
    acc += jnp.dot(mf_ref[...], pw_ref[...], preferred_element_type=jnp.float32)
    out_ref[...] = acc + b_ref[...]


def _tc_assemble(ga, gsm, mf, pa, psm, pw, b232):
    return pl.pallas_call(
        _tc_body,
        grid=(N // BR,),
        in_specs=[
            pl.BlockSpec((BR, ATOM_ID_DIM), lambda i: (i, 0)),
            pl.BlockSpec((BR, GDIM), lambda i: (i, 0)),
            pl.BlockSpec((BR, NUM_JOINS * MOTIF_FEAT_SIZE), lambda i: (i, 0)),
            pl.BlockSpec((ATOM_ID_DIM, OUT_DIM), lambda i: (0, 0)),
            pl.BlockSpec((GDIM, OUT_DIM), lambda i: (0, 0)),
            pl.BlockSpec((NUM_JOINS * MOTIF_FEAT_SIZE, OUT_DIM), lambda i: (0, 0)),
            pl.BlockSpec((1, OUT_DIM), lambda i: (0, 0)),
        ],
        out_specs=pl.BlockSpec((BR, OUT_DIM), lambda i: (i, 0)),
        out_shape=jax.ShapeDtypeStruct((N, OUT_DIM), jnp.float32),
        compiler_params=pltpu.CompilerParams(
            dimension_semantics=("arbitrary",),
        ),
    )(ga, gsm, mf, pa, psm, pw, b232)


def kernel(atom_idx, atom_charges, motif_features, shape_classes, mult_per_atom,
           atom_id_table, atom_charge_table, shape_id_table, atom_mult_table,
           W_motif, b_motif):
    f32 = jnp.float32
    i32 = jnp.int32
    ctab16 = jnp.zeros((3, 16), f32).at[:, :CHARGE_DIM].set(atom_charge_table)
    smtab = jnp.concatenate([
        jnp.tile(ctab16, (CREP, 1)),
        shape_id_table,
        jnp.tile(atom_mult_table, (MREP, 1)),
    ], axis=0)

    def rep8(col):
        return jnp.broadcast_to(col[:, None], (N, NSLOT)).reshape(-1)

    s_of_t = jax.lax.iota(i32, NSLOT * N) & (NSLOT - 1)
    rid = jnp.arange(N, dtype=i32)
    cterm = atom_charges.astype(i32) + 1 + 3 * (rid & (CREP - 1))
    mofs = (1 + MULT_OFS) + 32 * (rid & (MREP - 1))
    idx8 = jnp.where(s_of_t < 2, rep8(cterm), 0)
    for j in range(NUM_JOINS):
        idx8 += jnp.where(s_of_t == 2 + j,
                          rep8(shape_classes[:, j].astype(i32) + (1 + SHAPE_OFS)), 0)
        idx8 += jnp.where(s_of_t == 5 + j,
                          rep8(mult_per_atom[:, j].astype(i32) + mofs), 0)

    ga, gsm8 = _sc_gather(atom_idx.astype(i32), idx8, atom_id_table, smtab)
    gsm = gsm8.reshape(N, GDIM)

    pa = jnp.zeros((ATOM_ID_DIM, OUT_DIM), f32)
    pa = pa.at[0:32, 0:32].set(jnp.eye(32, dtype=f32))

    psm = jnp.zeros((GDIM, OUT_DIM), f32)
    psm = psm.at[0:CHARGE_DIM, 32:40].set(jnp.eye(CHARGE_DIM, dtype=f32))
    eye48 = jnp.eye(48, dtype=f32)
    psm = psm.at[32:80, 136:184].set(eye48)
    psm = psm.at[80:128, 184:232].set(eye48)

    pw = jnp.zeros((NUM_JOINS * MOTIF_FEAT_SIZE, OUT_DIM), f32)
    for j in range(NUM_JOINS):
        pw = pw.at[j * MOTIF_FEAT_SIZE:(j + 1) * MOTIF_FEAT_SIZE,
                   40 + j * MOTIF_DIM:40 + (j + 1) * MOTIF_DIM].set(W_motif)

    b232 = jnp.zeros((1, OUT_DIM), f32)
    b232 = b232.at[0, 40:136].set(jnp.tile(b_motif, NUM_JOINS))

    return _tc_assemble(ga, gsm, motif_features, pa, psm, pw, b232)


# speedup vs baseline: 1.9287x; 1.9287x over previous
import functools

import jax
import jax.numpy as jnp
from jax import lax
from jax.experimental import pallas as pl
from jax.experimental.pallas import tpu as pltpu
from jax.experimental.pallas import tpu_sc as plsc

N = 100000
ATOM_ID_DIM = 32
CHARGE_DIM = 8
MOTIF_FEAT_SIZE = 48
MOTIF_DIM = 32
NUM_JOINS = 3
OUT_DIM = 232
NSLOT = 8
GDIM = NSLOT * 16

CREP = 2048
MREP = 256
SHAPE_OFS = 3 * CREP
MULT_OFS = SHAPE_OFS + 5001


CHUNK = 1000
NUM_CHUNKS = N // CHUNK
CHALF = CHUNK // 2


def _sc_gather(aidx, idx8, atab, smtab):
    info = plsc.get_sparse_core_info()
    nc, ns = info.num_cores, info.num_subcores
    nw = nc * ns
    chunks_per_w = -(-NUM_CHUNKS // nw)
    mesh = plsc.VectorSubcoreMesh(core_axis_name="c", subcore_axis_name="s")

    @functools.partial(
        pl.kernel,
        mesh=mesh,
        out_type=(
            jax.ShapeDtypeStruct((N, ATOM_ID_DIM), jnp.float32),
            jax.ShapeDtypeStruct((NSLOT * N, 16), jnp.float32),
        ),
        scratch_types=[
            pltpu.VMEM((CHUNK,), jnp.int32),
            pltpu.VMEM((NSLOT * CHALF,), jnp.int32),
            pltpu.VMEM((CHUNK, ATOM_ID_DIM), jnp.float32),
            pltpu.VMEM((NSLOT * CHALF, 16), jnp.float32),
            pltpu.SemaphoreType.DMA,
        ],
        compiler_params=pltpu.CompilerParams(use_tc_tiling_on_sc=False),
    )
    def k(aidx_hbm, idx8_hbm, atab_hbm, smtab_hbm, ga_out, gsm_out,
          aidx_v, idx_v, arows_v, grows_v, sem):
        wid = lax.axis_index("s") * nc + lax.axis_index("c")
        for c in range(chunks_per_w):
            cid = wid + nw * c

            @pl.when(cid < NUM_CHUNKS)
            def _():
                base = cid * CHUNK
                rows = pl.ds(base, CHUNK)
                gr0 = pl.ds(base * NSLOT, NSLOT * CHALF)
                gr1 = pl.ds((base + CHALF) * NSLOT, NSLOT * CHALF)
                pltpu.sync_copy(aidx_hbm.at[rows], aidx_v)
                pltpu.sync_copy(idx8_hbm.at[gr0], idx_v)
                cp_a = pltpu.async_copy(atab_hbm.at[aidx_v], arows_v, sem)
                cp_g = pltpu.async_copy(smtab_hbm.at[idx_v], grows_v, sem)
                cp_a.wait()
                cp_g.wait()
                pltpu.sync_copy(arows_v, ga_out.at[rows])
                pltpu.sync_copy(grows_v, gsm_out.at[gr0])
                pltpu.sync_copy(idx8_hbm.at[gr1], idx_v)
                pltpu.async_copy(smtab_hbm.at[idx_v], grows_v, sem).wait()
                pltpu.sync_copy(grows_v, gsm_out.at[gr1])

    return k(aidx, idx8, atab, smtab)


BR = 1000


def _tc_body(ga_ref, gsm_ref, mf_ref, pa_ref, psm_ref, pw_ref, b_ref, out_ref):
    acc = jnp.dot(ga_ref[...], pa_ref[...], preferred_element_type=jnp.float32)
    acc += jnp.dot(gsm_ref[...], psm_ref[...], preferred_element_type=jnp.float32)
    acc += jnp.dot(mf_ref[...], pw_ref[...], preferred_element_type=jnp.float32)
    out_ref[...] = acc + b_ref[...]


def _tc_assemble(ga, gsm, mf, pa, psm, pw, b232):
    return pl.pallas_call(
        _tc_body,
        grid=(N // BR,),
        in_specs=[
            pl.BlockSpec((BR, ATOM_ID_DIM), lambda i: (i, 0)),
            pl.BlockSpec((BR, GDIM), lambda i: (i, 0)),
            pl.BlockSpec((BR, NUM_JOINS * MOTIF_FEAT_SIZE), lambda i: (i, 0)),
            pl.BlockSpec((ATOM_ID_DIM, OUT_DIM), lambda i: (0, 0)),
            pl.BlockSpec((GDIM, OUT_DIM), lambda i: (0, 0)),
            pl.BlockSpec((NUM_JOINS * MOTIF_FEAT_SIZE, OUT_DIM), lambda i: (0, 0)),
            pl.BlockSpec((1, OUT_DIM), lambda i: (0, 0)),
        ],
        out_specs=pl.BlockSpec((BR, OUT_DIM), lambda i: (i, 0)),
        out_shape=jax.ShapeDtypeStruct((N, OUT_DIM), jnp.float32),
        compiler_params=pltpu.CompilerParams(
            dimension_semantics=("arbitrary",),
        ),
    )(ga, gsm, mf, pa, psm, pw, b232)


def kernel(atom_idx, atom_charges, motif_features, shape_classes, mult_per_atom,
           atom_id_table, atom_charge_table, shape_id_table, atom_mult_table,
           W_motif, b_motif):
    f32 = jnp.float32
    i32 = jnp.int32
    ctab16 = jnp.zeros((3, 16), f32).at[:, :CHARGE_DIM].set(atom_charge_table)
    smtab = jnp.concatenate([
        jnp.tile(ctab16, (CREP, 1)),
        shape_id_table,
        jnp.tile(atom_mult_table, (MREP, 1)),
    ], axis=0)

    rid = jnp.arange(N, dtype=i32)
    c1 = atom_charges.astype(i32) + 1 + 3 * (rid & (CREP - 1))
    midx = (mult_per_atom.astype(i32) + (1 + MULT_OFS)
            + 32 * (rid & (MREP - 1))[:, None])
    idx8 = jnp.concatenate([
        c1[:, None], c1[:, None],
        shape_classes.astype(i32) + (1 + SHAPE_OFS),
        midx,
    ], axis=1).reshape(-1)

    ga, gsm8 = _sc_gather(atom_idx.astype(i32), idx8, atom_id_table, smtab)
    gsm = gsm8.reshape(N, GDIM)

    pa = jnp.zeros((ATOM_ID_DIM, OUT_DIM), f32)
    pa = pa.at[0:32, 0:32].set(jnp.eye(32, dtype=f32))

    psm = jnp.zeros((GDIM, OUT_DIM), f32)
    psm = psm.at[0:CHARGE_DIM, 32:40].set(jnp.eye(CHARGE_DIM, dtype=f32))
    eye48 = jnp.eye(48, dtype=f32)
    psm = psm.at[32:80, 136:184].set(eye48)
    psm = psm.at[80:128, 184:232].set(eye48)

    pw = jnp.zeros((NUM_JOINS * MOTIF_FEAT_SIZE, OUT_DIM), f32)
    for j in range(NUM_JOINS):
        pw = pw.at[j * MOTIF_FEAT_SIZE:(j + 1) * MOTIF_FEAT_SIZE,
                   40 + j * MOTIF_DIM:40 + (j + 1) * MOTIF_DIM].set(W_motif)

    b232 = jnp.zeros((1, OUT_DIM), f32)
    b232 = b232.at[0, 40:136].set(jnp.tile(b_motif, NUM_JOINS))

    return _tc_assemble(ga, gsm, motif_features, pa, psm, pw, b232)
